# baseline (speedup 1.0000x reference)
import functools
import jax
import jax.numpy as jnp
from jax import lax
from jax.experimental import pallas as pl
from jax.experimental.pallas import tpu as pltpu
from jax.experimental.pallas import tpu_sc as plsc

N = 10000
E = 320000
H = 128
IN_EDGE = 16
NP = 10240
NB = 1280
EP = 327680
EB = 4096
NTILES = 32
EPT = EP // NTILES
C1 = 128


def _sc_gather_body(pa_hbm, pb_hbm, xq_hbm, row_hbm, col_hbm,
                    ga_hbm, gb_hbm, xr_hbm, xc_hbm,
                    idx_r, idx_c, ga_v, gb_v, xr_v, xc_v,
                    sem_a, sem_b, sem_x):
    wid = lax.axis_index("s") * 2 + lax.axis_index("c")
    tile_base = wid * EPT

    def chunk(i, _):
        base = tile_base + i * C1
        pltpu.sync_copy(row_hbm.at[pl.ds(base, C1)], idx_r)
        pltpu.sync_copy(col_hbm.at[pl.ds(base, C1)], idx_c)
        da = pltpu.async_copy(pa_hbm.at[idx_r], ga_v, sem_a)
        db = pltpu.async_copy(pb_hbm.at[idx_c], gb_v, sem_b)
        dxr = pltpu.async_copy(xq_hbm.at[idx_r], xr_v, sem_x)
        dxc = pltpu.async_copy(xq_hbm.at[idx_c], xc_v, sem_x)
        dxr.wait()
        dxc.wait()
        pltpu.sync_copy(xr_v, xr_hbm.at[pl.ds(base, C1)])
        pltpu.sync_copy(xc_v, xc_hbm.at[pl.ds(base, C1)])
        da.wait()
        db.wait()
        pltpu.sync_copy(ga_v, ga_hbm.at[pl.ds(base, C1)])
        pltpu.sync_copy(gb_v, gb_hbm.at[pl.ds(base, C1)])
        return ()

    lax.fori_loop(0, EPT // C1, chunk, ())


def _sc_gather(pa, pb, xq, rowp, colp):
    mesh = plsc.VectorSubcoreMesh(core_axis_name="c", subcore_axis_name="s")
    f = pl.kernel(
        _sc_gather_body,
        compiler_params=pltpu.CompilerParams(use_tc_tiling_on_sc=False),
        out_type=[jax.ShapeDtypeStruct((EP, H // 2), jnp.float32),
                  jax.ShapeDtypeStruct((EP, H // 2), jnp.float32),
                  jax.ShapeDtypeStruct((EP, 16), jnp.float32),
                  jax.ShapeDtypeStruct((EP, 16), jnp.float32)],
        mesh=mesh,
        scratch_types=[
            pltpu.VMEM((C1,), jnp.int32),
            pltpu.VMEM((C1,), jnp.int32),
            pltpu.VMEM((C1, H // 2), jnp.float32),
            pltpu.VMEM((C1, H // 2), jnp.float32),
            pltpu.VMEM((C1, 16), jnp.float32),
            pltpu.VMEM((C1, 16), jnp.float32),
            pltpu.SemaphoreType.DMA,
            pltpu.SemaphoreType.DMA,
            pltpu.SemaphoreType.DMA,
        ],
    )
    return f(pa, pb, xq, rowp, colp)


def _pack_bf16_pair(v):
    lo = lax.bitcast_convert_type(
        v[:, :H // 2].astype(jnp.bfloat16), jnp.uint16).astype(jnp.uint32)
    hi = lax.bitcast_convert_type(
        v[:, H // 2:].astype(jnp.bfloat16), jnp.uint16).astype(jnp.uint32)
    return lax.bitcast_convert_type(lo | (hi << 16), jnp.float32)


def _unpack_bf16_pair(p):
    u = lax.bitcast_convert_type(p, jnp.uint32)
    lo = lax.bitcast_convert_type(u << 16, jnp.float32)
    hi = lax.bitcast_convert_type(u & jnp.uint32(0xFFFF0000), jnp.float32)
    return jnp.concatenate([lo, hi], axis=1)


def _node_pre_body(h_ref, Win_ref, bin_ref, Wa_ref, Wb_ref,
                   h0_ref, pa_ref, pb_ref):
    h0 = jnp.dot(h_ref[...], Win_ref[...], preferred_element_type=jnp.float32)
    h0 = h0 + bin_ref[...]
    h0_ref[...] = h0
    pa_ref[...] = _pack_bf16_pair(
        jnp.dot(h0, Wa_ref[...], preferred_element_type=jnp.float32))
    pb_ref[...] = _pack_bf16_pair(
        jnp.dot(h0, Wb_ref[...], preferred_element_type=jnp.float32))


def _node_pre(hp, W_in, b_in, Wa, Wb):
    grid = NP // NB
    blk = lambda i: (i, 0)
    wspec = pl.BlockSpec((H, H), lambda i: (0, 0))
    return pl.pallas_call(
        _node_pre_body,
        grid=(grid,),
        in_specs=[pl.BlockSpec((NB, H), blk), wspec,
                  pl.BlockSpec((1, H), lambda i: (0, 0)), wspec, wspec],
        out_specs=[pl.BlockSpec((NB, H), blk),
                   pl.BlockSpec((NB, H // 2), blk),
                   pl.BlockSpec((NB, H // 2), blk)],
        out_shape=[jax.ShapeDtypeStruct((NP, H), jnp.float32),
                   jax.ShapeDtypeStruct((NP, H // 2), jnp.float32),
                   jax.ShapeDtypeStruct((NP, H // 2), jnp.float32)],
    )(hp, W_in, b_in.reshape(1, H), Wa, Wb)


def _edge_mlp_body(ga_ref, gb_ref, ea_ref, xr_ref, xc_ref,
                   w1r_ref, We_ref, be1_ref, We2_ref, be2_ref,
                   Wc1_ref, bc1_ref, Wc2_ref, bc2_ref,
                   m2_ref, tr_ref):
    rel = xr_ref[...] - xc_ref[...]
    rad = jnp.sum(rel * rel, axis=1, keepdims=True)
    pre = _unpack_bf16_pair(ga_ref[...]) + _unpack_bf16_pair(gb_ref[...])
    pre = pre + rad * w1r_ref[...]
    pre = pre + jnp.dot(ea_ref[...], We_ref[...],
                        preferred_element_type=jnp.float32)
    m1 = jnp.maximum(pre + be1_ref[...], 0.0)
    bf = jnp.bfloat16
    m2 = jnp.maximum(
        jnp.dot(m1.astype(bf), We2_ref[...].astype(bf),
                preferred_element_type=jnp.float32)
        + be2_ref[...], 0.0)
    m2_ref[...] = m2
    c1 = jnp.maximum(
        jnp.dot(m2.astype(bf), Wc1_ref[...].astype(bf),
                preferred_element_type=jnp.float32)
        + bc1_ref[...], 0.0)
    cw = jnp.dot(c1.astype(bf), Wc2_ref[...].astype(bf),
                 preferred_element_type=jnp.float32)
    cw = cw[:, 0:1] + bc2_ref[...]
    tr_ref[...] = rel * cw


def _edge_mlp(ga, gb, ea, xr, xc, w1r, We1e, be1, We2, be2,
              Wc1, bc1, Wc2, bc2):
    grid = EP // EB
    blk = lambda i: (i, 0)
    c0 = lambda i: (0, 0)
    return pl.pallas_call(
        _edge_mlp_body,
        grid=(grid,),
        in_specs=[
            pl.BlockSpec((EB, H // 2), blk), pl.BlockSpec((EB, H // 2), blk),
            pl.BlockSpec((EB, IN_EDGE), blk),
            pl.BlockSpec((EB, 16), blk), pl.BlockSpec((EB, 16), blk),
            pl.BlockSpec((1, H), c0), pl.BlockSpec((IN_EDGE, H), c0),
            pl.BlockSpec((1, H), c0), pl.BlockSpec((H, H), c0),
            pl.BlockSpec((1, H), c0), pl.BlockSpec((H, H), c0),
            pl.BlockSpec((1, H), c0), pl.BlockSpec((H, 8), c0),
            pl.BlockSpec((1, 1), c0),
        ],
        out_specs=[pl.BlockSpec((EB, H), blk), pl.BlockSpec((EB, 16), blk)],
        out_shape=[jax.ShapeDtypeStruct((EP, H), jnp.float32),
                   jax.ShapeDtypeStruct((EP, 16), jnp.float32)],
    )(ga, gb, ea, xr, xc, w1r, We1e, be1, We2, be2,
      Wc1, bc1, Wc2, bc2)


C2 = 128


def _sc_scatter_body(m2_hbm, tr_hbm, row_hbm, agg_hbm, xacc_hbm,
                     idx, m2_v, tr_v, agg_sp, xacc_sp):
    cid = lax.axis_index("c")
    sid = lax.axis_index("s")
    wid = sid * 2 + cid
    tile_base = wid * EPT
    z = jnp.zeros((16,), jnp.float32)

    def zrow(r, _):
        for k in range(H // 16):
            m2_v[r, pl.ds(k * 16, 16)] = z
        tr_v[r, pl.ds(0, 16)] = z
        return ()

    lax.fori_loop(0, C2, zrow, ())
    for k in range(5):
        rows = sid * 640 + k * C2
        pltpu.sync_copy(m2_v, agg_sp.at[pl.ds(rows, C2)])
        pltpu.sync_copy(tr_v, xacc_sp.at[pl.ds(rows, C2)])
    plsc.subcore_barrier()

    def chunk(i, _):
        base = tile_base + i * C2
        pltpu.sync_copy(row_hbm.at[pl.ds(base, C2)], idx)
        pltpu.sync_copy(m2_hbm.at[pl.ds(base, C2)], m2_v)
        pltpu.sync_copy(tr_hbm.at[pl.ds(base, C2)], tr_v)
        pltpu.sync_copy(m2_v, agg_sp.at[idx], add=True)
        pltpu.sync_copy(tr_v, xacc_sp.at[idx], add=True)
        return ()

    lax.fori_loop(0, EPT // C2, chunk, ())
    plsc.subcore_barrier()

    for k in range(5):
        rows = sid * 640 + k * C2
        pltpu.sync_copy(agg_sp.at[pl.ds(rows, C2)], m2_v)
        pltpu.sync_copy(m2_v, agg_hbm.at[cid, pl.ds(rows, C2)])
        pltpu.sync_copy(xacc_sp.at[pl.ds(rows, C2)], tr_v)
        pltpu.sync_copy(tr_v, xacc_hbm.at[cid, pl.ds(rows, C2)])


def _sc_scatter(m2, tr, rowp):
    mesh = plsc.VectorSubcoreMesh(core_axis_name="c", subcore_axis_name="s")
    f = pl.kernel(
        _sc_scatter_body,
        compiler_params=pltpu.CompilerParams(use_tc_tiling_on_sc=False),
        out_type=[jax.ShapeDtypeStruct((2, NP, H), jnp.float32),
                  jax.ShapeDtypeStruct((2, NP, 16), jnp.float32)],
        mesh=mesh,
        scratch_types=[
            pltpu.VMEM((C2,), jnp.int32),
            pltpu.VMEM((C2, H), jnp.float32),
            pltpu.VMEM((C2, 16), jnp.float32),
            pltpu.VMEM_SHARED((NP, H), jnp.float32),
            pltpu.VMEM_SHARED((NP, 16), jnp.float32),
        ],
    )
    return f(m2, tr, rowp)


def _node_out_body(h0_ref, agg_ref, agg1_ref, xacc_ref, xacc1_ref, xp_ref,
                   Wna_ref, Wnb_ref, bn1_ref, Wn2_ref, bn2_ref,
                   Wo_ref, bo_ref, hout_ref, xout_ref):
    h0 = h0_ref[...]
    agg = agg_ref[...] + agg1_ref[...]
    t = jnp.dot(h0, Wna_ref[...], preferred_element_type=jnp.float32)
    t = t + jnp.dot(agg, Wnb_ref[...], preferred_element_type=jnp.float32)
    t = jnp.maximum(t + bn1_ref[...], 0.0)
    nh = jnp.dot(t, Wn2_ref[...], preferred_element_type=jnp.float32)
    h1 = h0 + nh + bn2_ref[...]
    hout_ref[...] = jnp.dot(h1, Wo_ref[...],
                            preferred_element_type=jnp.float32) + bo_ref[...]
    xout_ref[...] = xp_ref[...] + xacc_ref[...] + xacc1_ref[...]


def _node_out(h0, agg, agg1, xacc, xacc1, xp, Wna, Wnb, bn1, Wn2, bn2,
              Wo, bo):
    grid = NP // NB
    blk = lambda i: (i, 0)
    c0 = lambda i: (0, 0)
    wspec = pl.BlockSpec((H, H), c0)
    bspec = pl.BlockSpec((1, H), c0)
    return pl.pallas_call(
        _node_out_body,
        grid=(grid,),
        in_specs=[pl.BlockSpec((NB, H), blk), pl.BlockSpec((NB, H), blk),
                  pl.BlockSpec((NB, H), blk),
                  pl.BlockSpec((NB, 16), blk), pl.BlockSpec((NB, 16), blk),
                  pl.BlockSpec((NB, 16), blk),
                  wspec, wspec, bspec, wspec, bspec, wspec, bspec],
        out_specs=[pl.BlockSpec((NB, H), blk), pl.BlockSpec((NB, 16), blk)],
        out_shape=[jax.ShapeDtypeStruct((NP, H), jnp.float32),
                   jax.ShapeDtypeStruct((NP, 16), jnp.float32)],
    )(h0, agg, agg1, xacc, xacc1, xp, Wna, Wnb, bn1, Wn2, bn2, Wo, bo)


def kernel(h, x, edges, edge_attr, W_in, b_in, We1, be1, We2, be2,
           Wc1, bc1, Wc2, bc2, Wn1, bn1, Wn2, bn2, W_out, b_out):
    row = edges[0].astype(jnp.int32)
    col = edges[1].astype(jnp.int32)
    pad_idx = N + (jnp.arange(EP - E, dtype=jnp.int32) % (NP - N))
    rowp = jnp.concatenate([row, pad_idx])
    colp = jnp.concatenate([col, pad_idx])
    eap = jnp.pad(edge_attr, ((0, EP - E), (0, 0)))

    hp = jnp.pad(h, ((0, NP - N), (0, 0)))
    h0p, pa, pb = _node_pre(hp, W_in, b_in, We1[:H], We1[H:2 * H])

    xq = jnp.pad(x, ((0, NP - N), (0, 13)))
    ga, gb, xr, xc = _sc_gather(pa, pb, xq, rowp, colp)

    w1r = We1[2 * H].reshape(1, H)
    We1e = We1[2 * H + 1:]
    Wc2p = jnp.pad(Wc2, ((0, 0), (0, 7)))
    m2, tr = _edge_mlp(ga, gb, eap, xr, xc, w1r, We1e,
                       be1.reshape(1, H), We2, be2.reshape(1, H),
                       Wc1, bc1.reshape(1, H), Wc2p, bc2.reshape(1, 1))

    aggp, xaccp = _sc_scatter(m2, tr, rowp)

    xp = jnp.pad(x, ((0, NP - N), (0, 13)))
    hout, xout = _node_out(h0p, aggp[0], aggp[1], xaccp[0], xaccp[1], xp,
                           Wn1[:H], Wn1[H:],
                           bn1.reshape(1, H), Wn2, bn2.reshape(1, H),
                           W_out, b_out.reshape(1, H))
    return hout[:N], xout[:N, :3]


# speedup vs baseline: 1.4194x; 1.4194x over previous
import functools
import jax
import jax.numpy as jnp
from jax import lax
from jax.experimental import pallas as pl
from jax.experimental.pallas import tpu as pltpu
from jax.experimental.pallas import tpu_sc as plsc

N = 10000
E = 320000
H = 128
IN_EDGE = 16
NP = 10240
NB = 1280
EB = 4000
NTILES = 32
C1 = 128
NCHUNKS = E // C1


def _pack_bf16_pair(v):
    lo = lax.bitcast_convert_type(
        v[:, :H // 2].astype(jnp.bfloat16), jnp.uint16).astype(jnp.uint32)
    hi = lax.bitcast_convert_type(
        v[:, H // 2:].astype(jnp.bfloat16), jnp.uint16).astype(jnp.uint32)
    return lax.bitcast_convert_type(lo | (hi << 16), jnp.float32)


def _unpack_bf16_pair(p):
    u = lax.bitcast_convert_type(p, jnp.uint32)
    lo = lax.bitcast_convert_type(u << 16, jnp.float32)
    hi = lax.bitcast_convert_type(u & jnp.uint32(0xFFFF0000), jnp.float32)
    return jnp.concatenate([lo, hi], axis=1)


def _node_pre_body(h_ref, Win_ref, bin_ref, Wa_ref, Wb_ref,
                   h0_ref, pa_ref, pb_ref):
    h0 = jnp.dot(h_ref[...], Win_ref[...], preferred_element_type=jnp.float32)
    h0 = h0 + bin_ref[...]
    h0_ref[...] = h0
    pa_ref[...] = _pack_bf16_pair(
        jnp.dot(h0, Wa_ref[...], preferred_element_type=jnp.float32))
    pb_ref[...] = _pack_bf16_pair(
        jnp.dot(h0, Wb_ref[...], preferred_element_type=jnp.float32))


def _node_pre(hp, W_in, b_in, Wa, Wb):
    grid = NP // NB
    blk = lambda i: (i, 0)
    wspec = pl.BlockSpec((H, H), lambda i: (0, 0))
    return pl.pallas_call(
        _node_pre_body,
        grid=(grid,),
        in_specs=[pl.BlockSpec((NB, H), blk), wspec,
                  pl.BlockSpec((1, H), lambda i: (0, 0)), wspec, wspec],
        out_specs=[pl.BlockSpec((NB, H), blk),
                   pl.BlockSpec((NB, H // 2), blk),
                   pl.BlockSpec((NB, H // 2), blk)],
        out_shape=[jax.ShapeDtypeStruct((NP, H), jnp.float32),
                   jax.ShapeDtypeStruct((NP, H // 2), jnp.float32),
                   jax.ShapeDtypeStruct((NP, H // 2), jnp.float32)],
    )(hp, W_in, b_in.reshape(1, H), Wa, Wb)


def _sc_gather_body(pa_hbm, pb_hbm, xq_hbm, row_hbm, col_hbm,
                    gab_hbm, rel_hbm,
                    idx_r, idx_c, ga_v, gb_v, xr_v, xc_v, rel_v,
                    sem_a, sem_b, sem_x):
    wid = lax.axis_index("s") * 2 + lax.axis_index("c")
    nchunk = jnp.where(wid < NCHUNKS % NTILES,
                       NCHUNKS // NTILES + 1, NCHUNKS // NTILES)

    def chunk(i, _):
        base = (i * NTILES + wid) * C1
        pltpu.sync_copy(row_hbm.at[pl.ds(base, C1)], idx_r)
        pltpu.sync_copy(col_hbm.at[pl.ds(base, C1)], idx_c)
        da = pltpu.async_copy(pa_hbm.at[idx_r], ga_v, sem_a)
        db = pltpu.async_copy(pb_hbm.at[idx_c], gb_v, sem_b)
        dxr = pltpu.async_copy(xq_hbm.at[idx_r], xr_v, sem_x)
        dxc = pltpu.async_copy(xq_hbm.at[idx_c], xc_v, sem_x)
        dxr.wait()
        dxc.wait()
        for r in range(C1):
            rel_v[r, pl.ds(0, 16)] = (xr_v[r, pl.ds(0, 16)]
                                      - xc_v[r, pl.ds(0, 16)])
        pltpu.sync_copy(rel_v, rel_hbm.at[pl.ds(base, C1)])
        da.wait()
        db.wait()
        pltpu.sync_copy(ga_v, gab_hbm.at[pl.ds(base, C1), pl.ds(0, H // 2)])
        pltpu.sync_copy(gb_v,
                        gab_hbm.at[pl.ds(base, C1), pl.ds(H // 2, H // 2)])
        return ()

    lax.fori_loop(0, nchunk, chunk, ())


def _sc_gather(pa, pb, xq, row, col):
    mesh = plsc.VectorSubcoreMesh(core_axis_name="c", subcore_axis_name="s")
    f = pl.kernel(
        _sc_gather_body,
        compiler_params=pltpu.CompilerParams(use_tc_tiling_on_sc=False),
        out_type=[jax.ShapeDtypeStruct((E, H), jnp.float32),
                  jax.ShapeDtypeStruct((E, 16), jnp.float32)],
        mesh=mesh,
        scratch_types=[
            pltpu.VMEM((C1,), jnp.int32),
            pltpu.VMEM((C1,), jnp.int32),
            pltpu.VMEM((C1, H // 2), jnp.float32),
            pltpu.VMEM((C1, H // 2), jnp.float32),
            pltpu.VMEM((C1, 16), jnp.float32),
            pltpu.VMEM((C1, 16), jnp.float32),
            pltpu.VMEM((C1, 16), jnp.float32),
            pltpu.SemaphoreType.DMA,
            pltpu.SemaphoreType.DMA,
            pltpu.SemaphoreType.DMA,
        ],
    )
    return f(pa, pb, xq, row, col)


def _edge_mlp_body(gab_ref, ea_ref, rel_ref,
                   w1r_ref, We_ref, be1_ref, We2_ref, be2_ref,
                   Wc1_ref, bc1_ref, Wc2_ref, bc2_ref,
                   m2_ref, tr_ref):
    rel = rel_ref[...]
    rad = jnp.sum(rel * rel, axis=1, keepdims=True)
    gab = gab_ref[...]
    pre = (_unpack_bf16_pair(gab[:, :H // 2])
           + _unpack_bf16_pair(gab[:, H // 2:]))
    pre = pre + rad * w1r_ref[...]
    pre = pre + jnp.dot(ea_ref[...], We_ref[...],
                        preferred_element_type=jnp.float32)
    m1 = jnp.maximum(pre + be1_ref[...], 0.0)
    bf = jnp.bfloat16
    m2 = jnp.maximum(
        jnp.dot(m1.astype(bf), We2_ref[...].astype(bf),
                preferred_element_type=jnp.float32)
        + be2_ref[...], 0.0)
    m2_ref[...] = m2
    c1 = jnp.maximum(
        jnp.dot(m2.astype(bf), Wc1_ref[...].astype(bf),
                preferred_element_type=jnp.float32)
        + bc1_ref[...], 0.0)
    cw = jnp.dot(c1.astype(bf), Wc2_ref[...].astype(bf),
                 preferred_element_type=jnp.float32)
    cw = cw[:, 0:1] + bc2_ref[...]
    tr_ref[...] = rel * cw


def _edge_mlp(gab, ea, rel, w1r, We1e, be1, We2, be2, Wc1, bc1, Wc2, bc2):
    grid = E // EB
    blk = lambda i: (i, 0)
    c0 = lambda i: (0, 0)
    return pl.pallas_call(
        _edge_mlp_body,
        grid=(grid,),
        in_specs=[
            pl.BlockSpec((EB, H), blk),
            pl.BlockSpec((EB, IN_EDGE), blk),
            pl.BlockSpec((EB, 16), blk),
            pl.BlockSpec((1, H), c0), pl.BlockSpec((IN_EDGE, H), c0),
            pl.BlockSpec((1, H), c0), pl.BlockSpec((H, H), c0),
            pl.BlockSpec((1, H), c0), pl.BlockSpec((H, H), c0),
            pl.BlockSpec((1, H), c0), pl.BlockSpec((H, 8), c0),
            pl.BlockSpec((1, 1), c0),
        ],
        out_specs=[pl.BlockSpec((EB, H), blk), pl.BlockSpec((EB, 16), blk)],
        out_shape=[jax.ShapeDtypeStruct((E, H), jnp.float32),
                   jax.ShapeDtypeStruct((E, 16), jnp.float32)],
    )(gab, ea, rel, w1r, We1e, be1, We2, be2, Wc1, bc1, Wc2, bc2)


C2 = 128


def _sc_scatter_body(m2_hbm, tr_hbm, row_hbm, agg_hbm, xacc_hbm,
                     idx, m2_v, tr_v, agg_sp, xacc_sp):
    cid = lax.axis_index("c")
    sid = lax.axis_index("s")
    wid = sid * 2 + cid
    nchunk = jnp.where(wid < NCHUNKS % NTILES,
                       NCHUNKS // NTILES + 1, NCHUNKS // NTILES)
    z = jnp.zeros((16,), jnp.float32)

    def zrow(r, _):
        for k in range(H // 16):
            m2_v[r, pl.ds(k * 16, 16)] = z
        tr_v[r, pl.ds(0, 16)] = z
        return ()

    lax.fori_loop(0, C2, zrow, ())
    for k in range(5):
        rows = sid * 640 + k * C2
        pltpu.sync_copy(m2_v, agg_sp.at[pl.ds(rows, C2)])
        pltpu.sync_copy(tr_v, xacc_sp.at[pl.ds(rows, C2)])
    plsc.subcore_barrier()

    def chunk(i, _):
        base = (i * NTILES + wid) * C2
        pltpu.sync_copy(row_hbm.at[pl.ds(base, C2)], idx)
        pltpu.sync_copy(m2_hbm.at[pl.ds(base, C2)], m2_v)
        pltpu.sync_copy(tr_hbm.at[pl.ds(base, C2)], tr_v)
        pltpu.sync_copy(m2_v, agg_sp.at[idx], add=True)
        pltpu.sync_copy(tr_v, xacc_sp.at[idx], add=True)
        return ()

    lax.fori_loop(0, nchunk, chunk, ())
    plsc.subcore_barrier()

    for k in range(5):
        rows = sid * 640 + k * C2
        pltpu.sync_copy(agg_sp.at[pl.ds(rows, C2)], m2_v)
        pltpu.sync_copy(m2_v, agg_hbm.at[cid, pl.ds(rows, C2)])
        pltpu.sync_copy(xacc_sp.at[pl.ds(rows, C2)], tr_v)
        pltpu.sync_copy(tr_v, xacc_hbm.at[cid, pl.ds(rows, C2)])


def _sc_scatter(m2, tr, row):
    mesh = plsc.VectorSubcoreMesh(core_axis_name="c", subcore_axis_name="s")
    f = pl.kernel(
        _sc_scatter_body,
        compiler_params=pltpu.CompilerParams(use_tc_tiling_on_sc=False),
        out_type=[jax.ShapeDtypeStruct((2, NP, H), jnp.float32),
                  jax.ShapeDtypeStruct((2, NP, 16), jnp.float32)],
        mesh=mesh,
        scratch_types=[
            pltpu.VMEM((C2,), jnp.int32),
            pltpu.VMEM((C2, H), jnp.float32),
            pltpu.VMEM((C2, 16), jnp.float32),
            pltpu.VMEM_SHARED((NP, H), jnp.float32),
            pltpu.VMEM_SHARED((NP, 16), jnp.float32),
        ],
    )
    return f(m2, tr, row)


def _node_out_body(h0_ref, agg_ref, agg1_ref, xacc_ref, xacc1_ref, xp_ref,
                   Wna_ref, Wnb_ref, bn1_ref, Wn2_ref, bn2_ref,
                   Wo_ref, bo_ref, hout_ref, xout_ref):
    h0 = h0_ref[...]
    agg = agg_ref[...] + agg1_ref[...]
    t = jnp.dot(h0, Wna_ref[...], preferred_element_type=jnp.float32)
    t = t + jnp.dot(agg, Wnb_ref[...], preferred_element_type=jnp.float32)
    t = jnp.maximum(t + bn1_ref[...], 0.0)
    nh = jnp.dot(t, Wn2_ref[...], preferred_element_type=jnp.float32)
    h1 = h0 + nh + bn2_ref[...]
    hout_ref[...] = jnp.dot(h1, Wo_ref[...],
                            preferred_element_type=jnp.float32) + bo_ref[...]
    xout_ref[...] = xp_ref[...] + xacc_ref[...] + xacc1_ref[...]


def _node_out(h0, agg, agg1, xacc, xacc1, xp, Wna, Wnb, bn1, Wn2, bn2,
              Wo, bo):
    grid = NP // NB
    blk = lambda i: (i, 0)
    c0 = lambda i: (0, 0)
    wspec = pl.BlockSpec((H, H), c0)
    bspec = pl.BlockSpec((1, H), c0)
    return pl.pallas_call(
        _node_out_body,
        grid=(grid,),
        in_specs=[pl.BlockSpec((NB, H), blk), pl.BlockSpec((NB, H), blk),
                  pl.BlockSpec((NB, H), blk),
                  pl.BlockSpec((NB, 16), blk), pl.BlockSpec((NB, 16), blk),
                  pl.BlockSpec((NB, 16), blk),
                  wspec, wspec, bspec, wspec, bspec, wspec, bspec],
        out_specs=[pl.BlockSpec((NB, H), blk), pl.BlockSpec((NB, 16), blk)],
        out_shape=[jax.ShapeDtypeStruct((NP, H), jnp.float32),
                   jax.ShapeDtypeStruct((NP, 16), jnp.float32)],
    )(h0, agg, agg1, xacc, xacc1, xp, Wna, Wnb, bn1, Wn2, bn2, Wo, bo)


def kernel(h, x, edges, edge_attr, W_in, b_in, We1, be1, We2, be2,
           Wc1, bc1, Wc2, bc2, Wn1, bn1, Wn2, bn2, W_out, b_out):
    row = edges[0].astype(jnp.int32)
    col = edges[1].astype(jnp.int32)

    hp = jnp.pad(h, ((0, NP - N), (0, 0)))
    h0p, pa, pb = _node_pre(hp, W_in, b_in, We1[:H], We1[H:2 * H])

    xq = jnp.pad(x, ((0, NP - N), (0, 13)))
    gab, rel = _sc_gather(pa, pb, xq, row, col)

    w1r = We1[2 * H].reshape(1, H)
    We1e = We1[2 * H + 1:]
    Wc2p = jnp.pad(Wc2, ((0, 0), (0, 7)))
    m2, tr = _edge_mlp(gab, edge_attr, rel, w1r, We1e,
                       be1.reshape(1, H), We2, be2.reshape(1, H),
                       Wc1, bc1.reshape(1, H), Wc2p, bc2.reshape(1, 1))

    aggp, xaccp = _sc_scatter(m2, tr, row)

    xp = jnp.pad(x, ((0, NP - N), (0, 13)))
    hout, xout = _node_out(h0p, aggp[0], aggp[1], xaccp[0], xaccp[1], xp,
                           Wn1[:H], Wn1[H:],
                           bn1.reshape(1, H), Wn2, bn2.reshape(1, H),
                           W_out, b_out.reshape(1, H))
    return hout[:N], xout[:N, :3]


# speedup vs baseline: 1.5290x; 1.0773x over previous
import functools
import jax
import jax.numpy as jnp
from jax import lax
from jax.experimental import pallas as pl
from jax.experimental.pallas import tpu as pltpu
from jax.experimental.pallas import tpu_sc as plsc

N = 10000
E = 320000
H = 128
IN_EDGE = 16
NP = 10240
NB = 1280
EB = 4000
NTILES = 32
C1 = 128
NCHUNKS = E // C1


def _pack_bf16_pair(v):
    lo = lax.bitcast_convert_type(
        v[:, :H // 2].astype(jnp.bfloat16), jnp.uint16).astype(jnp.uint32)
    hi = lax.bitcast_convert_type(
        v[:, H // 2:].astype(jnp.bfloat16), jnp.uint16).astype(jnp.uint32)
    return lax.bitcast_convert_type(lo | (hi << 16), jnp.float32)


def _unpack_bf16_pair(p):
    u = lax.bitcast_convert_type(p, jnp.uint32)
    lo = lax.bitcast_convert_type(u << 16, jnp.float32)
    hi = lax.bitcast_convert_type(u & jnp.uint32(0xFFFF0000), jnp.float32)
    return jnp.concatenate([lo, hi], axis=1)


def _node_pre_body(h_ref, Win_ref, bin_ref, Wa_ref, Wb_ref,
                   h0_ref, pa_ref, pb_ref):
    h0 = jnp.dot(h_ref[...], Win_ref[...], preferred_element_type=jnp.float32)
    h0 = h0 + bin_ref[...]
    h0_ref[...] = h0
    pa_ref[...] = _pack_bf16_pair(
        jnp.dot(h0, Wa_ref[...], preferred_element_type=jnp.float32))
    pb_ref[...] = _pack_bf16_pair(
        jnp.dot(h0, Wb_ref[...], preferred_element_type=jnp.float32))


def _node_pre(hp, W_in, b_in, Wa, Wb):
    grid = NP // NB
    blk = lambda i: (i, 0)
    wspec = pl.BlockSpec((H, H), lambda i: (0, 0))
    return pl.pallas_call(
        _node_pre_body,
        grid=(grid,),
        in_specs=[pl.BlockSpec((NB, H), blk), wspec,
                  pl.BlockSpec((1, H), lambda i: (0, 0)), wspec, wspec],
        out_specs=[pl.BlockSpec((NB, H), blk),
                   pl.BlockSpec((NB, H // 2), blk),
                   pl.BlockSpec((NB, H // 2), blk)],
        out_shape=[jax.ShapeDtypeStruct((NP, H), jnp.float32),
                   jax.ShapeDtypeStruct((NP, H // 2), jnp.float32),
                   jax.ShapeDtypeStruct((NP, H // 2), jnp.float32)],
    )(hp, W_in, b_in.reshape(1, H), Wa, Wb)


NC1 = ((NCHUNKS + NTILES - 1) // NTILES + 1) // 2 * 2


def _sc_gather_body(pa_hbm, pb_hbm, xq_hbm, row_hbm, col_hbm,
                    gab_hbm, rel_hbm,
                    idx_r, idx_c, ga_v, gb_v, xr_v, xc_v, rel_v,
                    sem_i, sem_g, sem_w):
    wid = lax.axis_index("s") * 2 + lax.axis_index("c")

    def cbase(ch):
        return jnp.minimum(ch * NTILES + wid, NCHUNKS - 1) * C1

    def issue_idx(ch, b):
        base = cbase(ch)
        pltpu.async_copy(row_hbm.at[pl.ds(base, C1)], idx_r.at[b], sem_i[b])
        pltpu.async_copy(col_hbm.at[pl.ds(base, C1)], idx_c.at[b], sem_i[b])

    def drain_idx(b):
        pltpu.make_async_copy(row_hbm.at[pl.ds(0, C1)], idx_r.at[b],
                              sem_i[b]).wait()
        pltpu.make_async_copy(col_hbm.at[pl.ds(0, C1)], idx_c.at[b],
                              sem_i[b]).wait()

    def issue_gathers(b):
        pltpu.async_copy(pa_hbm.at[idx_r.at[b]], ga_v.at[b], sem_g[b])
        pltpu.async_copy(pb_hbm.at[idx_c.at[b]], gb_v.at[b], sem_g[b])
        pltpu.async_copy(xq_hbm.at[idx_r.at[b]], xr_v.at[b], sem_g[b])
        pltpu.async_copy(xq_hbm.at[idx_c.at[b]], xc_v.at[b], sem_g[b])

    def drain_gathers(b):
        pltpu.make_async_copy(pa_hbm.at[pl.ds(0, C1)], ga_v.at[b],
                              sem_g[b]).wait()
        pltpu.make_async_copy(pb_hbm.at[pl.ds(0, C1)], gb_v.at[b],
                              sem_g[b]).wait()
        pltpu.make_async_copy(xq_hbm.at[pl.ds(0, C1)], xr_v.at[b],
                              sem_g[b]).wait()
        pltpu.make_async_copy(xq_hbm.at[pl.ds(0, C1)], xc_v.at[b],
                              sem_g[b]).wait()

    def issue_writes(ch, b):
        base = cbase(ch)
        pltpu.async_copy(rel_v.at[b], rel_hbm.at[pl.ds(base, C1)], sem_w[b])
        pltpu.async_copy(ga_v.at[b],
                         gab_hbm.at[pl.ds(base, C1), pl.ds(0, H // 2)],
                         sem_w[b])
        pltpu.async_copy(gb_v.at[b],
                         gab_hbm.at[pl.ds(base, C1), pl.ds(H // 2, H // 2)],
                         sem_w[b])

    def drain_writes(b):
        pltpu.make_async_copy(rel_v.at[b], rel_hbm.at[pl.ds(0, C1)],
                              sem_w[b]).wait()
        pltpu.make_async_copy(ga_v.at[b],
                              gab_hbm.at[pl.ds(0, C1), pl.ds(0, H // 2)],
                              sem_w[b]).wait()
        pltpu.make_async_copy(gb_v.at[b],
                              gab_hbm.at[pl.ds(0, C1), pl.ds(H // 2, H // 2)],
                              sem_w[b]).wait()

    issue_idx(0, 0)
    issue_idx(1, 1)
    drain_idx(0)
    issue_gathers(0)

    def pair(i2, _):
        for b in (0, 1):
            ch = 2 * i2 + b
            drain_gathers(b)

            @pl.when(ch + 2 < NC1)
            def _():
                issue_idx(ch + 2, b)

            for r in range(C1):
                rel_v[b, r, pl.ds(0, 16)] = (xr_v[b, r, pl.ds(0, 16)]
                                             - xc_v[b, r, pl.ds(0, 16)])
            issue_writes(ch, b)

            @pl.when(ch + 1 < NC1)
            def _():
                drain_idx(1 - b)

                @pl.when(ch >= 1)
                def _():
                    drain_writes(1 - b)

                issue_gathers(1 - b)
        return ()

    lax.fori_loop(0, NC1 // 2, pair, ())
    drain_writes(0)
    drain_writes(1)


def _sc_gather(pa, pb, xq, row, col):
    mesh = plsc.VectorSubcoreMesh(core_axis_name="c", subcore_axis_name="s")
    f = pl.kernel(
        _sc_gather_body,
        compiler_params=pltpu.CompilerParams(use_tc_tiling_on_sc=False),
        out_type=[jax.ShapeDtypeStruct((E, H), jnp.float32),
                  jax.ShapeDtypeStruct((E, 16), jnp.float32)],
        mesh=mesh,
        scratch_types=[
            pltpu.VMEM((2, C1), jnp.int32),
            pltpu.VMEM((2, C1), jnp.int32),
            pltpu.VMEM((2, C1, H // 2), jnp.float32),
            pltpu.VMEM((2, C1, H // 2), jnp.float32),
            pltpu.VMEM((2, C1, 16), jnp.float32),
            pltpu.VMEM((2, C1, 16), jnp.float32),
            pltpu.VMEM((2, C1, 16), jnp.float32),
            [pltpu.SemaphoreType.DMA, pltpu.SemaphoreType.DMA],
            [pltpu.SemaphoreType.DMA, pltpu.SemaphoreType.DMA],
            [pltpu.SemaphoreType.DMA, pltpu.SemaphoreType.DMA],
        ],
    )
    return f(pa, pb, xq, row, col)


def _edge_mlp_body(gab_ref, ea_ref, rel_ref,
                   w1r_ref, We_ref, be1_ref, We2_ref, be2_ref,
                   Wc1_ref, bc1_ref, Wc2_ref, bc2_ref,
                   m2_ref, tr_ref):
    rel = rel_ref[...]
    rad = jnp.sum(rel * rel, axis=1, keepdims=True)
    gab = gab_ref[...]
    pre = (_unpack_bf16_pair(gab[:, :H // 2])
           + _unpack_bf16_pair(gab[:, H // 2:]))
    pre = pre + rad * w1r_ref[...]
    pre = pre + jnp.dot(ea_ref[...], We_ref[...],
                        preferred_element_type=jnp.float32)
    m1 = jnp.maximum(pre + be1_ref[...], 0.0)
    bf = jnp.bfloat16
    m2 = jnp.maximum(
        jnp.dot(m1.astype(bf), We2_ref[...].astype(bf),
                preferred_element_type=jnp.float32)
        + be2_ref[...], 0.0)
    m2_ref[...] = m2
    c1 = jnp.maximum(
        jnp.dot(m2.astype(bf), Wc1_ref[...].astype(bf),
                preferred_element_type=jnp.float32)
        + bc1_ref[...], 0.0)
    cw = jnp.dot(c1.astype(bf), Wc2_ref[...].astype(bf),
                 preferred_element_type=jnp.float32)
    cw = cw[:, 0:1] + bc2_ref[...]
    tr_ref[...] = rel * cw


def _edge_mlp(gab, ea, rel, w1r, We1e, be1, We2, be2, Wc1, bc1, Wc2, bc2):
    grid = E // EB
    blk = lambda i: (i, 0)
    c0 = lambda i: (0, 0)
    return pl.pallas_call(
        _edge_mlp_body,
        grid=(grid,),
        in_specs=[
            pl.BlockSpec((EB, H), blk),
            pl.BlockSpec((EB, IN_EDGE), blk),
            pl.BlockSpec((EB, 16), blk),
            pl.BlockSpec((1, H), c0), pl.BlockSpec((IN_EDGE, H), c0),
            pl.BlockSpec((1, H), c0), pl.BlockSpec((H, H), c0),
            pl.BlockSpec((1, H), c0), pl.BlockSpec((H, H), c0),
            pl.BlockSpec((1, H), c0), pl.BlockSpec((H, 8), c0),
            pl.BlockSpec((1, 1), c0),
        ],
        out_specs=[pl.BlockSpec((EB, H), blk), pl.BlockSpec((EB, 16), blk)],
        out_shape=[jax.ShapeDtypeStruct((E, H), jnp.float32),
                   jax.ShapeDtypeStruct((E, 16), jnp.float32)],
    )(gab, ea, rel, w1r, We1e, be1, We2, be2, Wc1, bc1, Wc2, bc2)


C2 = 128


def _sc_scatter_body(m2_hbm, tr_hbm, row_hbm, agg_hbm, xacc_hbm,
                     idx, m2_v, tr_v, agg_sp, xacc_sp):
    cid = lax.axis_index("c")
    sid = lax.axis_index("s")
    wid = sid * 2 + cid
    nchunk = jnp.where(wid < NCHUNKS % NTILES,
                       NCHUNKS // NTILES + 1, NCHUNKS // NTILES)
    z = jnp.zeros((16,), jnp.float32)

    def zrow(r, _):
        for k in range(H // 16):
            m2_v[r, pl.ds(k * 16, 16)] = z
        tr_v[r, pl.ds(0, 16)] = z
        return ()

    lax.fori_loop(0, C2, zrow, ())
    for k in range(5):
        rows = sid * 640 + k * C2
        pltpu.sync_copy(m2_v, agg_sp.at[pl.ds(rows, C2)])
        pltpu.sync_copy(tr_v, xacc_sp.at[pl.ds(rows, C2)])
    plsc.subcore_barrier()

    def chunk(i, _):
        base = (i * NTILES + wid) * C2
        pltpu.sync_copy(row_hbm.at[pl.ds(base, C2)], idx)
        pltpu.sync_copy(m2_hbm.at[pl.ds(base, C2)], m2_v)
        pltpu.sync_copy(tr_hbm.at[pl.ds(base, C2)], tr_v)
        pltpu.sync_copy(m2_v, agg_sp.at[idx], add=True)
        pltpu.sync_copy(tr_v, xacc_sp.at[idx], add=True)
        return ()

    lax.fori_loop(0, nchunk, chunk, ())
    plsc.subcore_barrier()

    for k in range(5):
        rows = sid * 640 + k * C2
        pltpu.sync_copy(agg_sp.at[pl.ds(rows, C2)], m2_v)
        pltpu.sync_copy(m2_v, agg_hbm.at[cid, pl.ds(rows, C2)])
        pltpu.sync_copy(xacc_sp.at[pl.ds(rows, C2)], tr_v)
        pltpu.sync_copy(tr_v, xacc_hbm.at[cid, pl.ds(rows, C2)])


def _sc_scatter(m2, tr, row):
    mesh = plsc.VectorSubcoreMesh(core_axis_name="c", subcore_axis_name="s")
    f = pl.kernel(
        _sc_scatter_body,
        compiler_params=pltpu.CompilerParams(use_tc_tiling_on_sc=False),
        out_type=[jax.ShapeDtypeStruct((2, NP, H), jnp.float32),
                  jax.ShapeDtypeStruct((2, NP, 16), jnp.float32)],
        mesh=mesh,
        scratch_types=[
            pltpu.VMEM((C2,), jnp.int32),
            pltpu.VMEM((C2, H), jnp.float32),
            pltpu.VMEM((C2, 16), jnp.float32),
            pltpu.VMEM_SHARED((NP, H), jnp.float32),
            pltpu.VMEM_SHARED((NP, 16), jnp.float32),
        ],
    )
    return f(m2, tr, row)


def _node_out_body(h0_ref, agg_ref, agg1_ref, xacc_ref, xacc1_ref, xp_ref,
                   Wna_ref, Wnb_ref, bn1_ref, Wn2_ref, bn2_ref,
                   Wo_ref, bo_ref, hout_ref, xout_ref):
    h0 = h0_ref[...]
    agg = agg_ref[...] + agg1_ref[...]
    t = jnp.dot(h0, Wna_ref[...], preferred_element_type=jnp.float32)
    t = t + jnp.dot(agg, Wnb_ref[...], preferred_element_type=jnp.float32)
    t = jnp.maximum(t + bn1_ref[...], 0.0)
    nh = jnp.dot(t, Wn2_ref[...], preferred_element_type=jnp.float32)
    h1 = h0 + nh + bn2_ref[...]
    hout_ref[...] = jnp.dot(h1, Wo_ref[...],
                            preferred_element_type=jnp.float32) + bo_ref[...]
    xout_ref[...] = xp_ref[...] + xacc_ref[...] + xacc1_ref[...]


def _node_out(h0, agg, agg1, xacc, xacc1, xp, Wna, Wnb, bn1, Wn2, bn2,
              Wo, bo):
    grid = NP // NB
    blk = lambda i: (i, 0)
    c0 = lambda i: (0, 0)
    wspec = pl.BlockSpec((H, H), c0)
    bspec = pl.BlockSpec((1, H), c0)
    return pl.pallas_call(
        _node_out_body,
        grid=(grid,),
        in_specs=[pl.BlockSpec((NB, H), blk), pl.BlockSpec((NB, H), blk),
                  pl.BlockSpec((NB, H), blk),
                  pl.BlockSpec((NB, 16), blk), pl.BlockSpec((NB, 16), blk),
                  pl.BlockSpec((NB, 16), blk),
                  wspec, wspec, bspec, wspec, bspec, wspec, bspec],
        out_specs=[pl.BlockSpec((NB, H), blk), pl.BlockSpec((NB, 16), blk)],
        out_shape=[jax.ShapeDtypeStruct((NP, H), jnp.float32),
                   jax.ShapeDtypeStruct((NP, 16), jnp.float32)],
    )(h0, agg, agg1, xacc, xacc1, xp, Wna, Wnb, bn1, Wn2, bn2, Wo, bo)


def kernel(h, x, edges, edge_attr, W_in, b_in, We1, be1, We2, be2,
           Wc1, bc1, Wc2, bc2, Wn1, bn1, Wn2, bn2, W_out, b_out):
    row = edges[0].astype(jnp.int32)
    col = edges[1].astype(jnp.int32)

    hp = jnp.pad(h, ((0, NP - N), (0, 0)))
    h0p, pa, pb = _node_pre(hp, W_in, b_in, We1[:H], We1[H:2 * H])

    xq = jnp.pad(x, ((0, NP - N), (0, 13)))
    gab, rel = _sc_gather(pa, pb, xq, row, col)

    w1r = We1[2 * H].reshape(1, H)
    We1e = We1[2 * H + 1:]
    Wc2p = jnp.pad(Wc2, ((0, 0), (0, 7)))
    m2, tr = _edge_mlp(gab, edge_attr, rel, w1r, We1e,
                       be1.reshape(1, H), We2, be2.reshape(1, H),
                       Wc1, bc1.reshape(1, H), Wc2p, bc2.reshape(1, 1))

    aggp, xaccp = _sc_scatter(m2, tr, row)

    xp = jnp.pad(x, ((0, NP - N), (0, 13)))
    hout, xout = _node_out(h0p, aggp[0], aggp[1], xaccp[0], xaccp[1], xp,
                           Wn1[:H], Wn1[H:],
                           bn1.reshape(1, H), Wn2, bn2.reshape(1, H),
                           W_out, b_out.reshape(1, H))
    return hout[:N], xout[:N, :3]


# speedup vs baseline: 1.7395x; 1.1376x over previous
import functools
import jax
import jax.numpy as jnp
from jax import lax
from jax.experimental import pallas as pl
from jax.experimental.pallas import tpu as pltpu
from jax.experimental.pallas import tpu_sc as plsc

N = 10000
E = 320000
H = 128
IN_EDGE = 16
NP = 10240
NB = 1280
EB = 4000
NTILES = 32
C1 = 128
NCHUNKS = E // C1


def _pack_bf16_pair(v):
    lo = lax.bitcast_convert_type(
        v[:, :H // 2].astype(jnp.bfloat16), jnp.uint16).astype(jnp.uint32)
    hi = lax.bitcast_convert_type(
        v[:, H // 2:].astype(jnp.bfloat16), jnp.uint16).astype(jnp.uint32)
    return lax.bitcast_convert_type(lo | (hi << 16), jnp.float32)


def _unpack_bf16_pair(p):
    u = lax.bitcast_convert_type(p, jnp.uint32)
    lo = lax.bitcast_convert_type(u << 16, jnp.float32)
    hi = lax.bitcast_convert_type(u & jnp.uint32(0xFFFF0000), jnp.float32)
    return jnp.concatenate([lo, hi], axis=1)


def _node_pre_body(h_ref, Win_ref, bin_ref, Wa_ref, Wb_ref,
                   h0_ref, pa_ref, pb_ref):
    h0 = jnp.dot(h_ref[...], Win_ref[...], preferred_element_type=jnp.float32)
    h0 = h0 + bin_ref[...]
    h0_ref[...] = h0
    pa_ref[...] = _pack_bf16_pair(
        jnp.dot(h0, Wa_ref[...], preferred_element_type=jnp.float32))
    pb_ref[...] = _pack_bf16_pair(
        jnp.dot(h0, Wb_ref[...], preferred_element_type=jnp.float32))


def _node_pre(hp, W_in, b_in, Wa, Wb):
    grid = NP // NB
    blk = lambda i: (i, 0)
    wspec = pl.BlockSpec((H, H), lambda i: (0, 0))
    return pl.pallas_call(
        _node_pre_body,
        grid=(grid,),
        in_specs=[pl.BlockSpec((NB, H), blk), wspec,
                  pl.BlockSpec((1, H), lambda i: (0, 0)), wspec, wspec],
        out_specs=[pl.BlockSpec((NB, H), blk),
                   pl.BlockSpec((NB, H // 2), blk),
                   pl.BlockSpec((NB, H // 2), blk)],
        out_shape=[jax.ShapeDtypeStruct((NP, H), jnp.float32),
                   jax.ShapeDtypeStruct((NP, H // 2), jnp.float32),
                   jax.ShapeDtypeStruct((NP, H // 2), jnp.float32)],
    )(hp, W_in, b_in.reshape(1, H), Wa, Wb)


NC1 = ((NCHUNKS + NTILES - 1) // NTILES + 1) // 2 * 2


def _sc_gather_body(pa_hbm, pb_hbm, xq_hbm, row_hbm, col_hbm,
                    gab_hbm, rel_hbm,
                    idx_r, idx_c, ga_v, gb_v, xr_v, xc_v, rel_v,
                    sem_i, sem_g, sem_w):
    wid = lax.axis_index("s") * 2 + lax.axis_index("c")

    def cbase(ch):
        return jnp.minimum(ch * NTILES + wid, NCHUNKS - 1) * C1

    def issue_idx(ch, b):
        base = cbase(ch)
        pltpu.async_copy(row_hbm.at[pl.ds(base, C1)], idx_r.at[b], sem_i[b])
        pltpu.async_copy(col_hbm.at[pl.ds(base, C1)], idx_c.at[b], sem_i[b])

    def drain_idx(b):
        pltpu.make_async_copy(row_hbm.at[pl.ds(0, C1)], idx_r.at[b],
                              sem_i[b]).wait()
        pltpu.make_async_copy(col_hbm.at[pl.ds(0, C1)], idx_c.at[b],
                              sem_i[b]).wait()

    def issue_gathers(b):
        pltpu.async_copy(pa_hbm.at[idx_r.at[b]], ga_v.at[b], sem_g[b])
        pltpu.async_copy(pb_hbm.at[idx_c.at[b]], gb_v.at[b], sem_g[b])
        pltpu.async_copy(xq_hbm.at[idx_r.at[b]], xr_v.at[b], sem_g[b])
        pltpu.async_copy(xq_hbm.at[idx_c.at[b]], xc_v.at[b], sem_g[b])

    def drain_gathers(b):
        pltpu.make_async_copy(pa_hbm.at[pl.ds(0, C1)], ga_v.at[b],
                              sem_g[b]).wait()
        pltpu.make_async_copy(pb_hbm.at[pl.ds(0, C1)], gb_v.at[b],
                              sem_g[b]).wait()
        pltpu.make_async_copy(xq_hbm.at[pl.ds(0, C1)], xr_v.at[b],
                              sem_g[b]).wait()
        pltpu.make_async_copy(xq_hbm.at[pl.ds(0, C1)], xc_v.at[b],
                              sem_g[b]).wait()

    def issue_writes(ch, b):
        base = cbase(ch)
        pltpu.async_copy(rel_v.at[b], rel_hbm.at[pl.ds(base, C1)], sem_w[b])
        pltpu.async_copy(ga_v.at[b],
                         gab_hbm.at[pl.ds(base, C1), pl.ds(0, H // 2)],
                         sem_w[b])
        pltpu.async_copy(gb_v.at[b],
                         gab_hbm.at[pl.ds(base, C1), pl.ds(H // 2, H // 2)],
                         sem_w[b])

    def drain_writes(b):
        pltpu.make_async_copy(rel_v.at[b], rel_hbm.at[pl.ds(0, C1)],
                              sem_w[b]).wait()
        pltpu.make_async_copy(ga_v.at[b],
                              gab_hbm.at[pl.ds(0, C1), pl.ds(0, H // 2)],
                              sem_w[b]).wait()
        pltpu.make_async_copy(gb_v.at[b],
                              gab_hbm.at[pl.ds(0, C1), pl.ds(H // 2, H // 2)],
                              sem_w[b]).wait()

    issue_idx(0, 0)
    issue_idx(1, 1)
    drain_idx(0)
    issue_gathers(0)

    def pair(i2, _):
        for b in (0, 1):
            ch = 2 * i2 + b
            drain_gathers(b)

            @pl.when(ch + 2 < NC1)
            def _():
                issue_idx(ch + 2, b)

            for r in range(C1):
                rel_v[b, r, pl.ds(0, 16)] = (xr_v[b, r, pl.ds(0, 16)]
                                             - xc_v[b, r, pl.ds(0, 16)])
            issue_writes(ch, b)

            @pl.when(ch + 1 < NC1)
            def _():
                drain_idx(1 - b)

                @pl.when(ch >= 1)
                def _():
                    drain_writes(1 - b)

                issue_gathers(1 - b)
        return ()

    lax.fori_loop(0, NC1 // 2, pair, ())
    drain_writes(0)
    drain_writes(1)


def _sc_gather(pa, pb, xq, row, col):
    mesh = plsc.VectorSubcoreMesh(core_axis_name="c", subcore_axis_name="s")
    f = pl.kernel(
        _sc_gather_body,
        compiler_params=pltpu.CompilerParams(use_tc_tiling_on_sc=False),
        out_type=[jax.ShapeDtypeStruct((E, H), jnp.float32),
                  jax.ShapeDtypeStruct((E, 16), jnp.float32)],
        mesh=mesh,
        scratch_types=[
            pltpu.VMEM((2, C1), jnp.int32),
            pltpu.VMEM((2, C1), jnp.int32),
            pltpu.VMEM((2, C1, H // 2), jnp.float32),
            pltpu.VMEM((2, C1, H // 2), jnp.float32),
            pltpu.VMEM((2, C1, 16), jnp.float32),
            pltpu.VMEM((2, C1, 16), jnp.float32),
            pltpu.VMEM((2, C1, 16), jnp.float32),
            [pltpu.SemaphoreType.DMA, pltpu.SemaphoreType.DMA],
            [pltpu.SemaphoreType.DMA, pltpu.SemaphoreType.DMA],
            [pltpu.SemaphoreType.DMA, pltpu.SemaphoreType.DMA],
        ],
    )
    return f(pa, pb, xq, row, col)


def _edge_mlp_body(gab_ref, ea_ref, rel_ref,
                   w1r_ref, We_ref, be1_ref, We2_ref, be2_ref,
                   Wc1_ref, bc1_ref, Wc2_ref, bc2_ref,
                   m2_ref, tr_ref):
    rel = rel_ref[...]
    rad = jnp.sum(rel * rel, axis=1, keepdims=True)
    gab = gab_ref[...]
    pre = (_unpack_bf16_pair(gab[:, :H // 2])
           + _unpack_bf16_pair(gab[:, H // 2:]))
    pre = pre + rad * w1r_ref[...]
    pre = pre + jnp.dot(ea_ref[...], We_ref[...],
                        preferred_element_type=jnp.float32)
    m1 = jnp.maximum(pre + be1_ref[...], 0.0)
    bf = jnp.bfloat16
    m2 = jnp.maximum(
        jnp.dot(m1.astype(bf), We2_ref[...].astype(bf),
                preferred_element_type=jnp.float32)
        + be2_ref[...], 0.0)
    m2_ref[...] = m2
    c1 = jnp.maximum(
        jnp.dot(m2.astype(bf), Wc1_ref[...].astype(bf),
                preferred_element_type=jnp.float32)
        + bc1_ref[...], 0.0)
    cw = jnp.dot(c1.astype(bf), Wc2_ref[...].astype(bf),
                 preferred_element_type=jnp.float32)
    cw = cw[:, 0:1] + bc2_ref[...]
    tr_ref[...] = rel * cw


def _edge_mlp(gab, ea, rel, w1r, We1e, be1, We2, be2, Wc1, bc1, Wc2, bc2):
    grid = E // EB
    blk = lambda i: (i, 0)
    c0 = lambda i: (0, 0)
    return pl.pallas_call(
        _edge_mlp_body,
        grid=(grid,),
        in_specs=[
            pl.BlockSpec((EB, H), blk),
            pl.BlockSpec((EB, IN_EDGE), blk),
            pl.BlockSpec((EB, 16), blk),
            pl.BlockSpec((1, H), c0), pl.BlockSpec((IN_EDGE, H), c0),
            pl.BlockSpec((1, H), c0), pl.BlockSpec((H, H), c0),
            pl.BlockSpec((1, H), c0), pl.BlockSpec((H, H), c0),
            pl.BlockSpec((1, H), c0), pl.BlockSpec((H, 8), c0),
            pl.BlockSpec((1, 1), c0),
        ],
        out_specs=[pl.BlockSpec((EB, H), blk), pl.BlockSpec((EB, 16), blk)],
        out_shape=[jax.ShapeDtypeStruct((E, H), jnp.float32),
                   jax.ShapeDtypeStruct((E, 16), jnp.float32)],
    )(gab, ea, rel, w1r, We1e, be1, We2, be2, Wc1, bc1, Wc2, bc2)


C2 = 128
EP = NC1 * NTILES * C2


def _sc_scatter_body(m2_hbm, tr_hbm, row_hbm, agg_hbm, xacc_hbm,
                     idx, m2_v, tr_v, agg_sp, xacc_sp, sem_l, sem_s):
    cid = lax.axis_index("c")
    sid = lax.axis_index("s")
    wid = sid * 2 + cid
    z = jnp.zeros((16,), jnp.float32)

    def zrow(r, _):
        for k in range(H // 16):
            m2_v[0, r, pl.ds(k * 16, 16)] = z
        tr_v[0, r, pl.ds(0, 16)] = z
        return ()

    lax.fori_loop(0, C2, zrow, ())
    for k in range(5):
        rows = sid * 640 + k * C2
        pltpu.sync_copy(m2_v.at[0], agg_sp.at[pl.ds(rows, C2)])
        pltpu.sync_copy(tr_v.at[0], xacc_sp.at[pl.ds(rows, C2)])
    plsc.subcore_barrier()

    def issue_loads(ch, b):
        ibase = (ch * NTILES + wid) * C2
        mbase = jnp.minimum(ch * NTILES + wid, NCHUNKS - 1) * C2
        pltpu.async_copy(row_hbm.at[pl.ds(ibase, C2)], idx.at[b], sem_l[b])
        pltpu.async_copy(m2_hbm.at[pl.ds(mbase, C2)], m2_v.at[b], sem_l[b])
        pltpu.async_copy(tr_hbm.at[pl.ds(mbase, C2)], tr_v.at[b], sem_l[b])

    def drain_loads(b):
        pltpu.make_async_copy(row_hbm.at[pl.ds(0, C2)], idx.at[b],
                              sem_l[b]).wait()
        pltpu.make_async_copy(m2_hbm.at[pl.ds(0, C2)], m2_v.at[b],
                              sem_l[b]).wait()
        pltpu.make_async_copy(tr_hbm.at[pl.ds(0, C2)], tr_v.at[b],
                              sem_l[b]).wait()

    def issue_adds(b):
        pltpu.async_copy(m2_v.at[b], agg_sp.at[idx.at[b]], sem_s[b],
                         add=True)
        pltpu.async_copy(tr_v.at[b], xacc_sp.at[idx.at[b]], sem_s[b],
                         add=True)

    def drain_adds(b):
        pltpu.make_async_copy(m2_v.at[b], agg_sp.at[idx.at[b]],
                              sem_s[b]).wait()
        pltpu.make_async_copy(tr_v.at[b], xacc_sp.at[idx.at[b]],
                              sem_s[b]).wait()

    issue_loads(0, 0)

    def pair(i2, _):
        for b in (0, 1):
            ch = 2 * i2 + b
            drain_loads(b)
            issue_adds(b)

            @pl.when(ch + 1 < NC1)
            def _():
                @pl.when(ch >= 1)
                def _():
                    drain_adds(1 - b)

                issue_loads(ch + 1, 1 - b)
        return ()

    lax.fori_loop(0, NC1 // 2, pair, ())
    drain_adds(0)
    drain_adds(1)
    plsc.subcore_barrier()

    for k in range(5):
        rows = sid * 640 + k * C2
        pltpu.sync_copy(agg_sp.at[pl.ds(rows, C2)], m2_v.at[0])
        pltpu.sync_copy(m2_v.at[0], agg_hbm.at[cid, pl.ds(rows, C2)])
        pltpu.sync_copy(xacc_sp.at[pl.ds(rows, C2)], tr_v.at[0])
        pltpu.sync_copy(tr_v.at[0], xacc_hbm.at[cid, pl.ds(rows, C2)])


def _sc_scatter(m2, tr, row2):
    mesh = plsc.VectorSubcoreMesh(core_axis_name="c", subcore_axis_name="s")
    f = pl.kernel(
        _sc_scatter_body,
        compiler_params=pltpu.CompilerParams(use_tc_tiling_on_sc=False),
        out_type=[jax.ShapeDtypeStruct((2, NP, H), jnp.float32),
                  jax.ShapeDtypeStruct((2, NP, 16), jnp.float32)],
        mesh=mesh,
        scratch_types=[
            pltpu.VMEM((2, C2), jnp.int32),
            pltpu.VMEM((2, C2, H), jnp.float32),
            pltpu.VMEM((2, C2, 16), jnp.float32),
            pltpu.VMEM_SHARED((NP, H), jnp.float32),
            pltpu.VMEM_SHARED((NP, 16), jnp.float32),
            [pltpu.SemaphoreType.DMA, pltpu.SemaphoreType.DMA],
            [pltpu.SemaphoreType.DMA, pltpu.SemaphoreType.DMA],
        ],
    )
    return f(m2, tr, row2)


def _node_out_body(h0_ref, agg_ref, agg1_ref, xacc_ref, xacc1_ref, xp_ref,
                   Wna_ref, Wnb_ref, bn1_ref, Wn2_ref, bn2_ref,
                   Wo_ref, bo_ref, hout_ref, xout_ref):
    h0 = h0_ref[...]
    agg = agg_ref[...] + agg1_ref[...]
    t = jnp.dot(h0, Wna_ref[...], preferred_element_type=jnp.float32)
    t = t + jnp.dot(agg, Wnb_ref[...], preferred_element_type=jnp.float32)
    t = jnp.maximum(t + bn1_ref[...], 0.0)
    nh = jnp.dot(t, Wn2_ref[...], preferred_element_type=jnp.float32)
    h1 = h0 + nh + bn2_ref[...]
    hout_ref[...] = jnp.dot(h1, Wo_ref[...],
                            preferred_element_type=jnp.float32) + bo_ref[...]
    xout_ref[...] = xp_ref[...] + xacc_ref[...] + xacc1_ref[...]


def _node_out(h0, agg, agg1, xacc, xacc1, xp, Wna, Wnb, bn1, Wn2, bn2,
              Wo, bo):
    grid = NP // NB
    blk = lambda i: (i, 0)
    c0 = lambda i: (0, 0)
    wspec = pl.BlockSpec((H, H), c0)
    bspec = pl.BlockSpec((1, H), c0)
    return pl.pallas_call(
        _node_out_body,
        grid=(grid,),
        in_specs=[pl.BlockSpec((NB, H), blk), pl.BlockSpec((NB, H), blk),
                  pl.BlockSpec((NB, H), blk),
                  pl.BlockSpec((NB, 16), blk), pl.BlockSpec((NB, 16), blk),
                  pl.BlockSpec((NB, 16), blk),
                  wspec, wspec, bspec, wspec, bspec, wspec, bspec],
        out_specs=[pl.BlockSpec((NB, H), blk), pl.BlockSpec((NB, 16), blk)],
        out_shape=[jax.ShapeDtypeStruct((NP, H), jnp.float32),
                   jax.ShapeDtypeStruct((NP, 16), jnp.float32)],
    )(h0, agg, agg1, xacc, xacc1, xp, Wna, Wnb, bn1, Wn2, bn2, Wo, bo)


def kernel(h, x, edges, edge_attr, W_in, b_in, We1, be1, We2, be2,
           Wc1, bc1, Wc2, bc2, Wn1, bn1, Wn2, bn2, W_out, b_out):
    row = edges[0].astype(jnp.int32)
    col = edges[1].astype(jnp.int32)

    hp = jnp.pad(h, ((0, NP - N), (0, 0)))
    h0p, pa, pb = _node_pre(hp, W_in, b_in, We1[:H], We1[H:2 * H])

    xq = jnp.pad(x, ((0, NP - N), (0, 13)))
    gab, rel = _sc_gather(pa, pb, xq, row, col)

    w1r = We1[2 * H].reshape(1, H)
    We1e = We1[2 * H + 1:]
    Wc2p = jnp.pad(Wc2, ((0, 0), (0, 7)))
    m2, tr = _edge_mlp(gab, edge_attr, rel, w1r, We1e,
                       be1.reshape(1, H), We2, be2.reshape(1, H),
                       Wc1, bc1.reshape(1, H), Wc2p, bc2.reshape(1, 1))

    pad_idx = N + (jnp.arange(EP - E, dtype=jnp.int32) % (NP - N))
    row2 = jnp.concatenate([row, pad_idx])
    aggp, xaccp = _sc_scatter(m2, tr, row2)

    xp = jnp.pad(x, ((0, NP - N), (0, 13)))
    hout, xout = _node_out(h0p, aggp[0], aggp[1], xaccp[0], xaccp[1], xp,
                           Wn1[:H], Wn1[H:],
                           bn1.reshape(1, H), Wn2, bn2.reshape(1, H),
                           W_out, b_out.reshape(1, H))
    return hout[:N], xout[:N, :3]


# speedup vs baseline: 1.7872x; 1.0275x over previous
import functools
import jax
import jax.numpy as jnp
from jax import lax
from jax.experimental import pallas as pl
from jax.experimental.pallas import tpu as pltpu
from jax.experimental.pallas import tpu_sc as plsc

N = 10000
E = 320000
H = 128
IN_EDGE = 16
NP = 10240
NB = 1280
EB = 8000
NTILES = 32
C1 = 128
NCHUNKS = E // C1


def _pack_bf16_pair(v):
    lo = lax.bitcast_convert_type(
        v[:, :H // 2].astype(jnp.bfloat16), jnp.uint16).astype(jnp.uint32)
    hi = lax.bitcast_convert_type(
        v[:, H // 2:].astype(jnp.bfloat16), jnp.uint16).astype(jnp.uint32)
    return lax.bitcast_convert_type(lo | (hi << 16), jnp.float32)


def _unpack_bf16_pair(p):
    u = lax.bitcast_convert_type(p, jnp.uint32)
    lo = lax.bitcast_convert_type(u << 16, jnp.float32)
    hi = lax.bitcast_convert_type(u & jnp.uint32(0xFFFF0000), jnp.float32)
    return jnp.concatenate([lo, hi], axis=1)


def _node_pre_body(h_ref, Win_ref, bin_ref, Wa_ref, Wb_ref,
                   h0_ref, pa_ref, pb_ref):
    h0 = jnp.dot(h_ref[...], Win_ref[...], preferred_element_type=jnp.float32)
    h0 = h0 + bin_ref[...]
    h0_ref[...] = h0
    pa_ref[...] = _pack_bf16_pair(
        jnp.dot(h0, Wa_ref[...], preferred_element_type=jnp.float32))
    pb_ref[...] = _pack_bf16_pair(
        jnp.dot(h0, Wb_ref[...], preferred_element_type=jnp.float32))


def _node_pre(hp, W_in, b_in, Wa, Wb):
    grid = NP // NB
    blk = lambda i: (i, 0)
    wspec = pl.BlockSpec((H, H), lambda i: (0, 0))
    return pl.pallas_call(
        _node_pre_body,
        grid=(grid,),
        in_specs=[pl.BlockSpec((NB, H), blk), wspec,
                  pl.BlockSpec((1, H), lambda i: (0, 0)), wspec, wspec],
        out_specs=[pl.BlockSpec((NB, H), blk),
                   pl.BlockSpec((NB, H // 2), blk),
                   pl.BlockSpec((NB, H // 2), blk)],
        out_shape=[jax.ShapeDtypeStruct((NP, H), jnp.float32),
                   jax.ShapeDtypeStruct((NP, H // 2), jnp.float32),
                   jax.ShapeDtypeStruct((NP, H // 2), jnp.float32)],
    )(hp, W_in, b_in.reshape(1, H), Wa, Wb)


NC1 = ((NCHUNKS + NTILES - 1) // NTILES + 1) // 2 * 2


def _sc_gather_body(pa_hbm, pb_hbm, xq_hbm, row_hbm, col_hbm,
                    gab_hbm, rel_hbm,
                    idx_r, idx_c, ga_v, gb_v, xr_v, xc_v, rel_v,
                    sem_i, sem_g, sem_w):
    wid = lax.axis_index("s") * 2 + lax.axis_index("c")

    def cbase(ch):
        return jnp.minimum(ch * NTILES + wid, NCHUNKS - 1) * C1

    def issue_idx(ch, b):
        base = cbase(ch)
        pltpu.async_copy(row_hbm.at[pl.ds(base, C1)], idx_r.at[b], sem_i[b])
        pltpu.async_copy(col_hbm.at[pl.ds(base, C1)], idx_c.at[b], sem_i[b])

    def drain_idx(b):
        pltpu.make_async_copy(row_hbm.at[pl.ds(0, C1)], idx_r.at[b],
                              sem_i[b]).wait()
        pltpu.make_async_copy(col_hbm.at[pl.ds(0, C1)], idx_c.at[b],
                              sem_i[b]).wait()

    def issue_gathers(b):
        pltpu.async_copy(pa_hbm.at[idx_r.at[b]], ga_v.at[b], sem_g[b])
        pltpu.async_copy(pb_hbm.at[idx_c.at[b]], gb_v.at[b], sem_g[b])
        pltpu.async_copy(xq_hbm.at[idx_r.at[b]], xr_v.at[b], sem_g[b])
        pltpu.async_copy(xq_hbm.at[idx_c.at[b]], xc_v.at[b], sem_g[b])

    def drain_gathers(b):
        pltpu.make_async_copy(pa_hbm.at[pl.ds(0, C1)], ga_v.at[b],
                              sem_g[b]).wait()
        pltpu.make_async_copy(pb_hbm.at[pl.ds(0, C1)], gb_v.at[b],
                              sem_g[b]).wait()
        pltpu.make_async_copy(xq_hbm.at[pl.ds(0, C1)], xr_v.at[b],
                              sem_g[b]).wait()
        pltpu.make_async_copy(xq_hbm.at[pl.ds(0, C1)], xc_v.at[b],
                              sem_g[b]).wait()

    def issue_writes(ch, b):
        base = cbase(ch)
        pltpu.async_copy(rel_v.at[b], rel_hbm.at[pl.ds(base, C1)], sem_w[b])
        pltpu.async_copy(ga_v.at[b],
                         gab_hbm.at[pl.ds(base, C1), pl.ds(0, H // 2)],
                         sem_w[b])
        pltpu.async_copy(gb_v.at[b],
                         gab_hbm.at[pl.ds(base, C1), pl.ds(H // 2, H // 2)],
                         sem_w[b])

    def drain_writes(b):
        pltpu.make_async_copy(rel_v.at[b], rel_hbm.at[pl.ds(0, C1)],
                              sem_w[b]).wait()
        pltpu.make_async_copy(ga_v.at[b],
                              gab_hbm.at[pl.ds(0, C1), pl.ds(0, H // 2)],
                              sem_w[b]).wait()
        pltpu.make_async_copy(gb_v.at[b],
                              gab_hbm.at[pl.ds(0, C1), pl.ds(H // 2, H // 2)],
                              sem_w[b]).wait()

    issue_idx(0, 0)
    issue_idx(1, 1)
    drain_idx(0)
    issue_gathers(0)

    def pair(i2, _):
        for b in (0, 1):
            ch = 2 * i2 + b
            drain_gathers(b)

            @pl.when(ch + 2 < NC1)
            def _():
                issue_idx(ch + 2, b)

            for r in range(C1):
                rel_v[b, r, pl.ds(0, 16)] = (xr_v[b, r, pl.ds(0, 16)]
                                             - xc_v[b, r, pl.ds(0, 16)])
            issue_writes(ch, b)

            @pl.when(ch + 1 < NC1)
            def _():
                drain_idx(1 - b)

                @pl.when(ch >= 1)
                def _():
                    drain_writes(1 - b)

                issue_gathers(1 - b)
        return ()

    lax.fori_loop(0, NC1 // 2, pair, ())
    drain_writes(0)
    drain_writes(1)


def _sc_gather(pa, pb, xq, row, col):
    mesh = plsc.VectorSubcoreMesh(core_axis_name="c", subcore_axis_name="s")
    f = pl.kernel(
        _sc_gather_body,
        compiler_params=pltpu.CompilerParams(use_tc_tiling_on_sc=False),
        out_type=[jax.ShapeDtypeStruct((E, H), jnp.float32),
                  jax.ShapeDtypeStruct((E, 16), jnp.float32)],
        mesh=mesh,
        scratch_types=[
            pltpu.VMEM((2, C1), jnp.int32),
            pltpu.VMEM((2, C1), jnp.int32),
            pltpu.VMEM((2, C1, H // 2), jnp.float32),
            pltpu.VMEM((2, C1, H // 2), jnp.float32),
            pltpu.VMEM((2, C1, 16), jnp.float32),
            pltpu.VMEM((2, C1, 16), jnp.float32),
            pltpu.VMEM((2, C1, 16), jnp.float32),
            [pltpu.SemaphoreType.DMA, pltpu.SemaphoreType.DMA],
            [pltpu.SemaphoreType.DMA, pltpu.SemaphoreType.DMA],
            [pltpu.SemaphoreType.DMA, pltpu.SemaphoreType.DMA],
        ],
    )
    return f(pa, pb, xq, row, col)


def _edge_mlp_body(gab_ref, ea_ref, rel_ref,
                   w1r_ref, We_ref, be1_ref, We2_ref, be2_ref,
                   Wc1_ref, bc1_ref, Wc2_ref, bc2_ref,
                   m2_ref, tr_ref):
    rel = rel_ref[...]
    rad = jnp.sum(rel * rel, axis=1, keepdims=True)
    gab = gab_ref[...]
    pre = (_unpack_bf16_pair(gab[:, :H // 2])
           + _unpack_bf16_pair(gab[:, H // 2:]))
    pre = pre + rad * w1r_ref[...]
    pre = pre + jnp.dot(ea_ref[...], We_ref[...],
                        preferred_element_type=jnp.float32)
    m1 = jnp.maximum(pre + be1_ref[...], 0.0)
    bf = jnp.bfloat16
    m2 = jnp.maximum(
        jnp.dot(m1.astype(bf), We2_ref[...].astype(bf),
                preferred_element_type=jnp.float32)
        + be2_ref[...], 0.0)
    m2_ref[...] = m2
    c1 = jnp.maximum(
        jnp.dot(m2.astype(bf), Wc1_ref[...].astype(bf),
                preferred_element_type=jnp.float32)
        + bc1_ref[...], 0.0)
    cw = jnp.dot(c1.astype(bf), Wc2_ref[...].astype(bf),
                 preferred_element_type=jnp.float32)
    cw = cw[:, 0:1] + bc2_ref[...]
    tr_ref[...] = rel * cw


def _edge_mlp(gab, ea, rel, w1r, We1e, be1, We2, be2, Wc1, bc1, Wc2, bc2):
    grid = E // EB
    blk = lambda i: (i, 0)
    c0 = lambda i: (0, 0)
    return pl.pallas_call(
        _edge_mlp_body,
        grid=(grid,),
        in_specs=[
            pl.BlockSpec((EB, H), blk),
            pl.BlockSpec((EB, IN_EDGE), blk),
            pl.BlockSpec((EB, 16), blk),
            pl.BlockSpec((1, H), c0), pl.BlockSpec((IN_EDGE, H), c0),
            pl.BlockSpec((1, H), c0), pl.BlockSpec((H, H), c0),
            pl.BlockSpec((1, H), c0), pl.BlockSpec((H, H), c0),
            pl.BlockSpec((1, H), c0), pl.BlockSpec((H, 8), c0),
            pl.BlockSpec((1, 1), c0),
        ],
        out_specs=[pl.BlockSpec((EB, H), blk), pl.BlockSpec((EB, 16), blk)],
        out_shape=[jax.ShapeDtypeStruct((E, H), jnp.float32),
                   jax.ShapeDtypeStruct((E, 16), jnp.float32)],
    )(gab, ea, rel, w1r, We1e, be1, We2, be2, Wc1, bc1, Wc2, bc2)


C2 = 128
EP = NC1 * NTILES * C2


def _sc_scatter_body(m2_hbm, tr_hbm, row_hbm, agg_hbm, xacc_hbm,
                     idx, m2_v, tr_v, agg_sp, xacc_sp, sem_l, sem_s):
    cid = lax.axis_index("c")
    sid = lax.axis_index("s")
    wid = sid * 2 + cid
    z = jnp.zeros((16,), jnp.float32)

    def zrow(r, _):
        for k in range(H // 16):
            m2_v[0, r, pl.ds(k * 16, 16)] = z
        tr_v[0, r, pl.ds(0, 16)] = z
        return ()

    lax.fori_loop(0, C2, zrow, ())
    for k in range(5):
        rows = sid * 640 + k * C2
        pltpu.sync_copy(m2_v.at[0], agg_sp.at[pl.ds(rows, C2)])
        pltpu.sync_copy(tr_v.at[0], xacc_sp.at[pl.ds(rows, C2)])
    plsc.subcore_barrier()

    def issue_loads(ch, b):
        ibase = (ch * NTILES + wid) * C2
        mbase = jnp.minimum(ch * NTILES + wid, NCHUNKS - 1) * C2
        pltpu.async_copy(row_hbm.at[pl.ds(ibase, C2)], idx.at[b], sem_l[b])
        pltpu.async_copy(m2_hbm.at[pl.ds(mbase, C2)], m2_v.at[b], sem_l[b])
        pltpu.async_copy(tr_hbm.at[pl.ds(mbase, C2)], tr_v.at[b], sem_l[b])

    def drain_loads(b):
        pltpu.make_async_copy(row_hbm.at[pl.ds(0, C2)], idx.at[b],
                              sem_l[b]).wait()
        pltpu.make_async_copy(m2_hbm.at[pl.ds(0, C2)], m2_v.at[b],
                              sem_l[b]).wait()
        pltpu.make_async_copy(tr_hbm.at[pl.ds(0, C2)], tr_v.at[b],
                              sem_l[b]).wait()

    def issue_adds(b):
        pltpu.async_copy(m2_v.at[b], agg_sp.at[idx.at[b]], sem_s[b],
                         add=True)
        pltpu.async_copy(tr_v.at[b], xacc_sp.at[idx.at[b]], sem_s[b],
                         add=True)

    def drain_adds(b):
        pltpu.make_async_copy(m2_v.at[b], agg_sp.at[idx.at[b]],
                              sem_s[b]).wait()
        pltpu.make_async_copy(tr_v.at[b], xacc_sp.at[idx.at[b]],
                              sem_s[b]).wait()

    issue_loads(0, 0)

    def pair(i2, _):
        for b in (0, 1):
            ch = 2 * i2 + b
            drain_loads(b)
            issue_adds(b)

            @pl.when(ch + 1 < NC1)
            def _():
                @pl.when(ch >= 1)
                def _():
                    drain_adds(1 - b)

                issue_loads(ch + 1, 1 - b)
        return ()

    lax.fori_loop(0, NC1 // 2, pair, ())
    drain_adds(0)
    drain_adds(1)
    plsc.subcore_barrier()

    for k in range(5):
        rows = sid * 640 + k * C2
        pltpu.sync_copy(agg_sp.at[pl.ds(rows, C2)], m2_v.at[0])
        pltpu.sync_copy(m2_v.at[0], agg_hbm.at[cid, pl.ds(rows, C2)])
        pltpu.sync_copy(xacc_sp.at[pl.ds(rows, C2)], tr_v.at[0])
        pltpu.sync_copy(tr_v.at[0], xacc_hbm.at[cid, pl.ds(rows, C2)])


def _sc_scatter(m2, tr, row2):
    mesh = plsc.VectorSubcoreMesh(core_axis_name="c", subcore_axis_name="s")
    f = pl.kernel(
        _sc_scatter_body,
        compiler_params=pltpu.CompilerParams(use_tc_tiling_on_sc=False),
        out_type=[jax.ShapeDtypeStruct((2, NP, H), jnp.float32),
                  jax.ShapeDtypeStruct((2, NP, 16), jnp.float32)],
        mesh=mesh,
        scratch_types=[
            pltpu.VMEM((2, C2), jnp.int32),
            pltpu.VMEM((2, C2, H), jnp.float32),
            pltpu.VMEM((2, C2, 16), jnp.float32),
            pltpu.VMEM_SHARED((NP, H), jnp.float32),
            pltpu.VMEM_SHARED((NP, 16), jnp.float32),
            [pltpu.SemaphoreType.DMA, pltpu.SemaphoreType.DMA],
            [pltpu.SemaphoreType.DMA, pltpu.SemaphoreType.DMA],
        ],
    )
    return f(m2, tr, row2)


def _node_out_body(h0_ref, agg_ref, agg1_ref, xacc_ref, xacc1_ref, xp_ref,
                   Wna_ref, Wnb_ref, bn1_ref, Wn2_ref, bn2_ref,
                   Wo_ref, bo_ref, hout_ref, xout_ref):
    h0 = h0_ref[...]
    agg = agg_ref[...] + agg1_ref[...]
    t = jnp.dot(h0, Wna_ref[...], preferred_element_type=jnp.float32)
    t = t + jnp.dot(agg, Wnb_ref[...], preferred_element_type=jnp.float32)
    t = jnp.maximum(t + bn1_ref[...], 0.0)
    nh = jnp.dot(t, Wn2_ref[...], preferred_element_type=jnp.float32)
    h1 = h0 + nh + bn2_ref[...]
    hout_ref[...] = jnp.dot(h1, Wo_ref[...],
                            preferred_element_type=jnp.float32) + bo_ref[...]
    xout_ref[...] = xp_ref[...] + xacc_ref[...] + xacc1_ref[...]


def _node_out(h0, agg, agg1, xacc, xacc1, xp, Wna, Wnb, bn1, Wn2, bn2,
              Wo, bo):
    grid = NP // NB
    blk = lambda i: (i, 0)
    c0 = lambda i: (0, 0)
    wspec = pl.BlockSpec((H, H), c0)
    bspec = pl.BlockSpec((1, H), c0)
    return pl.pallas_call(
        _node_out_body,
        grid=(grid,),
        in_specs=[pl.BlockSpec((NB, H), blk), pl.BlockSpec((NB, H), blk),
                  pl.BlockSpec((NB, H), blk),
                  pl.BlockSpec((NB, 16), blk), pl.BlockSpec((NB, 16), blk),
                  pl.BlockSpec((NB, 16), blk),
                  wspec, wspec, bspec, wspec, bspec, wspec, bspec],
        out_specs=[pl.BlockSpec((NB, H), blk), pl.BlockSpec((NB, 16), blk)],
        out_shape=[jax.ShapeDtypeStruct((NP, H), jnp.float32),
                   jax.ShapeDtypeStruct((NP, 16), jnp.float32)],
    )(h0, agg, agg1, xacc, xacc1, xp, Wna, Wnb, bn1, Wn2, bn2, Wo, bo)


def kernel(h, x, edges, edge_attr, W_in, b_in, We1, be1, We2, be2,
           Wc1, bc1, Wc2, bc2, Wn1, bn1, Wn2, bn2, W_out, b_out):
    row = edges[0].astype(jnp.int32)
    col = edges[1].astype(jnp.int32)

    hp = jnp.pad(h, ((0, NP - N), (0, 0)))
    h0p, pa, pb = _node_pre(hp, W_in, b_in, We1[:H], We1[H:2 * H])

    xq = jnp.pad(x, ((0, NP - N), (0, 13)))
    gab, rel = _sc_gather(pa, pb, xq, row, col)

    w1r = We1[2 * H].reshape(1, H)
    We1e = We1[2 * H + 1:]
    Wc2p = jnp.pad(Wc2, ((0, 0), (0, 7)))
    m2, tr = _edge_mlp(gab, edge_attr, rel, w1r, We1e,
                       be1.reshape(1, H), We2, be2.reshape(1, H),
                       Wc1, bc1.reshape(1, H), Wc2p, bc2.reshape(1, 1))

    pad_idx = N + (jnp.arange(EP - E, dtype=jnp.int32) % (NP - N))
    row2 = jnp.concatenate([row, pad_idx])
    aggp, xaccp = _sc_scatter(m2, tr, row2)

    xp = jnp.pad(x, ((0, NP - N), (0, 13)))
    hout, xout = _node_out(h0p, aggp[0], aggp[1], xaccp[0], xaccp[1], xp,
                           Wn1[:H], Wn1[H:],
                           bn1.reshape(1, H), Wn2, bn2.reshape(1, H),
                           W_out, b_out.reshape(1, H))
    return hout[:N], xout[:N, :3]
